# Initial kernel scaffold; baseline (speedup 1.0000x reference)
import jax
import jax.numpy as jnp
from jax.experimental import pallas as pl


def kernel(x, edge_index, batch, W0, b0, W1, b1, conv1_w, conv1_b, conv2_w, conv2_b):
    raise NotImplementedError("write your pallas kernel here")


# speedup vs baseline: 9.8892x; 9.8892x over previous
import functools

import jax
import jax.numpy as jnp
from jax import lax
from jax.experimental import pallas as pl
from jax.experimental.pallas import tpu as pltpu
from jax.experimental.pallas import tpu_sc as plsc

G = 16
RW = 128


def _sc_degree(dst_t, ones_rows, zrows, npad, ch, nc, ns):
  rows_pt = npad // ns
  mesh = plsc.VectorSubcoreMesh(core_axis_name="c", subcore_axis_name="s")

  @functools.partial(
      pl.kernel,
      out_type=jax.ShapeDtypeStruct((nc, npad, RW), jnp.float32),
      mesh=mesh,
      scratch_types=[
          pltpu.VMEM((128,), jnp.int32),
          pltpu.VMEM((128, RW), jnp.float32),
          pltpu.VMEM_SHARED((npad, RW), jnp.float32),
      ],
  )
  def deg_kernel(dst_hbm, ones_hbm, z_hbm, out_hbm, idx_v, ones_v, acc):
    c = lax.axis_index("c")
    s = lax.axis_index("s")
    wid = s * nc + c
    pltpu.sync_copy(ones_hbm, ones_v)
    pltpu.sync_copy(z_hbm, acc.at[pl.ds(s * rows_pt, rows_pt)])
    plsc.subcore_barrier()

    def body(j, carry):
      pltpu.sync_copy(dst_hbm.at[wid, j], idx_v)
      pltpu.sync_copy(ones_v, acc.at[idx_v], add=True)
      return carry

    lax.fori_loop(0, ch, body, 0)
    plsc.subcore_barrier()
    sl = pl.ds(s * rows_pt, rows_pt)
    pltpu.sync_copy(acc.at[sl], out_hbm.at[c, sl])

  return deg_kernel(dst_t, ones_rows, zrows)


def _sc_aggregate(z, src_t, dst_t, zrows, npad, ch, nc, ns):
  rows_pt = npad // ns
  mesh = plsc.VectorSubcoreMesh(core_axis_name="c", subcore_axis_name="s")

  @functools.partial(
      pl.kernel,
      out_type=jax.ShapeDtypeStruct((nc, npad, RW), jnp.float32),
      mesh=mesh,
      scratch_types=[
          pltpu.VMEM((128,), jnp.int32),
          pltpu.VMEM((128,), jnp.int32),
          pltpu.VMEM((128, RW), jnp.float32),
          pltpu.VMEM_SHARED((npad, RW), jnp.float32),
          pltpu.SemaphoreType.DMA,
      ],
  )
  def agg_kernel(z_hbm, src_hbm, dst_hbm, zr_hbm, out_hbm,
                 src_v, dst_v, rows_v, acc, sem):
    c = lax.axis_index("c")
    s = lax.axis_index("s")
    wid = s * nc + c
    pltpu.sync_copy(zr_hbm, acc.at[pl.ds(s * rows_pt, rows_pt)])
    plsc.subcore_barrier()

    def body(j, carry):
      pltpu.sync_copy(src_hbm.at[wid, j], src_v)
      pltpu.sync_copy(dst_hbm.at[wid, j], dst_v)
      pltpu.async_copy(z_hbm.at[src_v], rows_v, sem).wait()
      pltpu.sync_copy(rows_v, acc.at[dst_v], add=True)
      return carry

    lax.fori_loop(0, ch, body, 0)
    plsc.subcore_barrier()
    sl = pl.ds(s * rows_pt, rows_pt)
    pltpu.sync_copy(acc.at[sl], out_hbm.at[c, sl])

  return agg_kernel(z, src_t, dst_t, zrows)


def _tc_first(x, w0, degp, n):
  def body(x_ref, w_ref, d_ref, z_ref, dis_ref):
    deg = d_ref[0, :n, 0:1] + d_ref[1, :n, 0:1] + 1.0
    dis = lax.rsqrt(deg)
    dis_ref[...] = dis
    z = jnp.dot(x_ref[...], w_ref[...],
                preferred_element_type=jnp.float32) * dis
    z_ref[...] = jnp.concatenate(
        [z, jnp.zeros((n, RW - z.shape[1]), jnp.float32)], axis=1)

  return pl.pallas_call(
      body,
      out_shape=(jax.ShapeDtypeStruct((n, RW), jnp.float32),
                 jax.ShapeDtypeStruct((n, 1), jnp.float32)),
  )(x, w0, degp)


def _tc_mid(aggp, z1, dis, b0, w1, n, h):
  def body(a_ref, z_ref, dis_ref, b_ref, w_ref, z2_ref):
    agg = a_ref[0, :n, :h] + a_ref[1, :n, :h]
    h1 = jnp.maximum(
        dis_ref[...] * (agg + z_ref[:, :h]) + b_ref[...], 0.0)
    z2 = jnp.dot(h1, w_ref[...],
                 preferred_element_type=jnp.float32) * dis_ref[...]
    z2_ref[...] = jnp.concatenate(
        [z2, jnp.zeros((n, RW - h), jnp.float32)], axis=1)

  return pl.pallas_call(
      body,
      out_shape=jax.ShapeDtypeStruct((n, RW), jnp.float32),
  )(aggp, z1, dis, b0, w1)


def _tc_last(aggp, z2, dis, b1, n, h):
  def body(a_ref, z_ref, dis_ref, b_ref, o_ref):
    agg = a_ref[0, :n, :h] + a_ref[1, :n, :h]
    o_ref[...] = jnp.maximum(
        dis_ref[...] * (agg + z_ref[:, :h]) + b_ref[...], 0.0)

  return pl.pallas_call(
      body,
      out_shape=jax.ShapeDtypeStruct((n, h), jnp.float32),
  )(aggp, z2, dis, b1)


CT = 2000
CTE = CT + 4
F = 3


def _tc_cnn(scalars, xpad, w1t, w2t, bias, n, lseq, cin, h):
  nt = lseq // CT
  lp_half = CT // 2

  def body(s_ref, x_ref, w1_ref, w2_ref, b_ref, out_ref):
    g = pl.program_id(0)
    t = pl.program_id(1)
    o = s_ref[g]
    cnt = s_ref[g + 1] - o
    lp = s_ref[G + 1]

    q = lax.broadcasted_iota(jnp.int32, (CTE, 1), 0)
    p = t * CT - 2 + q

    acc = jnp.zeros((CTE, h), jnp.float32) + b_ref[0:1, :]
    for k in range(3):
      sl = x_ref[pl.ds(o + t * CT + k, CTE), :]
      tap = p + (k - 1)
      m = (tap >= 0) & (tap < cnt)
      acc = acc + jnp.dot(jnp.where(m, sl, 0.0), w1_ref[k],
                          preferred_element_type=jnp.float32)
    c1 = jnp.maximum(acc, 0.0)

    pe = jnp.max(c1.reshape(CTE // 2, 2, h), axis=1)
    jv = t * lp_half - 1 + lax.broadcasted_iota(jnp.int32, (lp_half + 2, 1), 0)
    pm = jnp.where((jv >= 0) & (jv < lp), pe, 0.0)

    c2 = jnp.maximum(
        jnp.dot(pm[0:lp_half], w2_ref[0], preferred_element_type=jnp.float32)
        + jnp.dot(pm[1:lp_half + 1], w2_ref[1],
                  preferred_element_type=jnp.float32)
        + jnp.dot(pm[2:lp_half + 2], w2_ref[2],
                  preferred_element_type=jnp.float32)
        + b_ref[1:2, :], 0.0)
    jc = t * lp_half + lax.broadcasted_iota(jnp.int32, (lp_half, 1), 0)
    c2m = jnp.where(jc < lp, c2, -jnp.inf)
    loc = jnp.max(c2m, axis=0, keepdims=True)[None]

    @pl.when(t == 0)
    def _():
      out_ref[...] = loc

    @pl.when(t > 0)
    def _():
      out_ref[...] = jnp.maximum(out_ref[...], loc)

  grid_spec = pltpu.PrefetchScalarGridSpec(
      num_scalar_prefetch=1,
      grid=(G, nt),
      in_specs=[
          pl.BlockSpec(xpad.shape, lambda g, t, s: (0, 0)),
          pl.BlockSpec(w1t.shape, lambda g, t, s: (0, 0, 0)),
          pl.BlockSpec(w2t.shape, lambda g, t, s: (0, 0, 0)),
          pl.BlockSpec(bias.shape, lambda g, t, s: (0, 0)),
      ],
      out_specs=pl.BlockSpec((1, 1, h), lambda g, t, s: (g, 0, 0)),
  )
  out = pl.pallas_call(
      body,
      grid_spec=grid_spec,
      out_shape=jax.ShapeDtypeStruct((G, 1, h), jnp.float32),
  )(scalars, xpad, w1t, w2t, bias)
  return out[:, 0, :]


def kernel(x, edge_index, batch, W0, b0, W1, b1,
           conv1_w, conv1_b, conv2_w, conv2_b):
  n, cin = x.shape
  h = W0.shape[1]
  e = edge_index.shape[1]
  info = plsc.get_sparse_core_info()
  nc, ns = info.num_cores, info.num_subcores
  nw = nc * ns

  ept = -(-e // nw)
  ch = -(-ept // 128)
  ept_pad = ch * 128
  npad = -(-(n + 1) // (8 * ns)) * (8 * ns)

  src = edge_index[0]
  dst = edge_index[1]
  total_pad = nw * ept_pad - e
  src_t = jnp.concatenate(
      [src, jnp.zeros((total_pad,), jnp.int32)]).reshape(nw, ch, 128)
  dst_t = jnp.concatenate(
      [dst, jnp.full((total_pad,), n, jnp.int32)]).reshape(nw, ch, 128)

  ones_rows = jnp.ones((128, RW), jnp.float32)
  zrows = jnp.zeros((npad // ns, RW), jnp.float32)

  degp = _sc_degree(dst_t, ones_rows, zrows, npad, ch, nc, ns)
  z1, dis = _tc_first(x, W0, degp, n)
  agg1 = _sc_aggregate(z1, src_t, dst_t, zrows, npad, ch, nc, ns)
  z2 = _tc_mid(agg1, z1, dis, b0.reshape(1, h), W1, n, h)
  agg2 = _sc_aggregate(z2, src_t, dst_t, zrows, npad, ch, nc, ns)
  h_out = _tc_last(agg2, z2, dis, b1.reshape(1, h), n, h)

  offsets = jnp.searchsorted(batch, jnp.arange(G, dtype=jnp.int32)
                             ).astype(jnp.int32)
  offs_full = jnp.concatenate([offsets, jnp.array([n], jnp.int32)])
  counts = offs_full[1:] - offs_full[:-1]
  lp = (counts.max() // 2).astype(jnp.int32)
  scalars = jnp.concatenate([offs_full, lp[None]])

  lseq = n
  nxp = F + n + lseq + 16
  nxp = -(-nxp // 8) * 8
  xpad = jnp.zeros((nxp, cin), jnp.float32).at[F:F + n].set(x)
  w1t = jnp.transpose(conv1_w, (2, 1, 0))
  w2t = jnp.transpose(conv2_w, (2, 1, 0))
  bias = jnp.stack([conv1_b, conv2_b])

  global_reps = _tc_cnn(scalars, xpad, w1t, w2t, bias, n, lseq, cin, h)
  return (h_out, global_reps)
